# baseline (speedup 1.0000x reference)
import jax
import jax.numpy as jnp
from jax.experimental import pallas as pl
from jax.experimental.pallas import tpu as pltpu

N_ENT = 16384
CB = 2048
NCHUNK = N_ENT // CB
NBUF = 3


def _dot_t(a, b):
    return jax.lax.dot_general(
        a, b, (((1,), (1,)), ((), ())), preferred_element_type=jnp.float32
    )


def _ln(v, w, b):
    mu = jnp.mean(v)
    var = jnp.mean((v - mu) ** 2)
    return (v - mu) / jnp.sqrt(var + 1e-5) * w + b


def _fused_kernel(
    em_ref, ar_ref, wk_ref, bk_ref, w0_ref, b0_ref, w1_ref, b1_ref,
    wf_ref, bf_ref, wi0_ref, bi0_ref, wi1_ref, bi1_ref, wo_ref, bo_ref,
    lnw_ref, lnb_ref, enc_hbm, unit_ref, targ_ref,
    b0buf, b1buf, b2buf, row_sc, q_sc, qb_sc, sems
):
    i = pl.program_id(0)
    bufs = (b0buf, b1buf, b2buf)

    def _chunk_copy(c, buf, k):
        return pltpu.make_async_copy(
            enc_hbm.at[pl.ds(c * CB, CB), :], buf, sems.at[k]
        )

    @pl.when(i == 0)
    def _prologue():
        _chunk_copy(0, b0buf, 0).start()
        _chunk_copy(1, b1buf, 1).start()
        ar = ar_ref[...]
        intermed = _dot_t(ar, w0_ref[...]) + b0_ref[...]
        intermed = jnp.maximum(
            _dot_t(jnp.maximum(intermed, 0.0), w1_ref[...]) + b1_ref[...], 0.0
        )
        x = jnp.concatenate([intermed, jnp.zeros_like(intermed)], axis=1)
        lnw = lnw_ref[...]
        lnb = lnb_ref[...]
        remember = _ln(
            jax.nn.sigmoid(_dot_t(x, wi0_ref[...]) + bi0_ref[...])
            * jnp.tanh(_dot_t(x, wi1_ref[...]) + bi1_ref[...]),
            lnw, lnb,
        )
        out_gate = _ln(jax.nn.sigmoid(_dot_t(x, wo_ref[...]) + bo_ref[...]), lnw, lnb)
        query = jnp.tanh(remember) * out_gate
        q256 = jax.lax.dot_general(
            query, wk_ref[...], (((1,), (0,)), ((), ())),
            preferred_element_type=jnp.float32,
        )
        q_sc[0:1, 0:256] = q256
        qb_sc[0] = jnp.sum(query * bk_ref[...])

    q256 = q_sc[0:1, 0:256]

    def _body(k):
        buf = bufs[k]
        _chunk_copy(i, buf, k).wait()

        @pl.when(i + 2 < NCHUNK)
        def _issue_next():
            _chunk_copy(i + 2, bufs[(k + 2) % NBUF], (k + 2) % NBUF).start()

        sim = jax.lax.dot_general(
            q256, buf[...], (((1,), (1,)), ((), ())),
            precision=jax.lax.Precision.HIGHEST,
            preferred_element_type=jnp.float32,
        ) + qb_sc[0]
        logit = jax.nn.sigmoid(sim)
        vec = jnp.exp(jnp.log(logit) / 0.8)
        row_sc[0:1, pl.ds(i * CB, CB)] = vec

    for k in range(NBUF):
        @pl.when(i % NBUF == k)
        def _run(k=k):
            _body(k)

    @pl.when(i == NCHUNK - 1)
    def _epilogue():
        vecrow = row_sc[...]
        s = jnp.sum(vecrow)
        bmax = jnp.max(vecrow)
        colf = jax.lax.broadcasted_iota(jnp.int32, (1, N_ENT), 1)
        pick = jnp.min(jnp.where(vecrow == bmax, colf, N_ENT))
        unit_ref[...] = jnp.where(s != 0.0, vecrow / s, vecrow)
        targ_ref[...] = jnp.where(
            (colf == pick) & jnp.logical_not(em_ref[...]), 1.0, 0.0
        )


def kernel(utype_mask, entity_mask, entity_encodings, autoregressive_encoding,
           self_unit_ct, W_keys, b_keys, W0, b0, W1, b1, Wf, bf, Wi0, bi0,
           Wi1, bi1, Wo, bo, ln_w, ln_b):
    em = entity_mask.reshape(1, N_ENT)
    ar2 = autoregressive_encoding.reshape(1, 1024)
    row = lambda v: v.reshape(1, -1)

    full = lambda shape: pl.BlockSpec(shape, lambda i: (0, 0))
    unit, targ = pl.pallas_call(
        _fused_kernel,
        grid=(NCHUNK,),
        in_specs=[
            full((1, N_ENT)),
            full((1, 1024)),
            full(W_keys.shape),
            full((1, 32)),
            full(W0.shape), full((1, 256)),
            full(W1.shape), full((1, 32)),
            full(Wf.shape), full((1, 32)),
            full(Wi0.shape), full((1, 32)),
            full(Wi1.shape), full((1, 32)),
            full(Wo.shape), full((1, 32)),
            full((1, 32)), full((1, 32)),
            pl.BlockSpec(memory_space=pltpu.MemorySpace.HBM),
        ],
        out_specs=[
            pl.BlockSpec((1, N_ENT), lambda i: (0, 0)),
            pl.BlockSpec((1, N_ENT), lambda i: (0, 0)),
        ],
        out_shape=[
            jax.ShapeDtypeStruct((1, N_ENT), jnp.float32),
            jax.ShapeDtypeStruct((1, N_ENT), jnp.float32),
        ],
        scratch_shapes=[
            pltpu.VMEM((CB, 256), jnp.float32),
            pltpu.VMEM((CB, 256), jnp.float32),
            pltpu.VMEM((CB, 256), jnp.float32),
            pltpu.VMEM((1, N_ENT), jnp.float32),
            pltpu.VMEM((8, 256), jnp.float32),
            pltpu.SMEM((1,), jnp.float32),
            pltpu.SemaphoreType.DMA((NBUF,)),
        ],
    )(
        em, ar2, W_keys, row(b_keys), W0, row(b0),
        W1, row(b1), Wf, row(bf), Wi0, row(bi0), Wi1, row(bi1),
        Wo, row(bo), row(ln_w), row(ln_b), entity_encodings
    )
    return unit, targ.reshape(N_ENT)


# speedup vs baseline: 1.7347x; 1.7347x over previous
import jax
import jax.numpy as jnp
from jax.experimental import pallas as pl
from jax.experimental.pallas import tpu as pltpu

N_ENT = 16384
CB = 2048
NCHUNK = N_ENT // CB
NBUF = 3


def _dot_t(a, b):
    return jax.lax.dot_general(
        a, b, (((1,), (1,)), ((), ())), preferred_element_type=jnp.float32
    )


def _ln(v, w, b):
    mu = jnp.mean(v)
    var = jnp.mean((v - mu) ** 2)
    return (v - mu) / jnp.sqrt(var + 1e-5) * w + b


def _fused_kernel(
    em_ref, ar_ref, wk_ref, bk_ref, w0_ref, b0_ref, w1_ref, b1_ref,
    wf_ref, bf_ref, wi0_ref, bi0_ref, wi1_ref, bi1_ref, wo_ref, bo_ref,
    lnw_ref, lnb_ref, enc_hbm, unit_ref, targ_ref,
    b0buf, b1buf, b2buf, row_sc, q_sc, qb_sc, sems
):
    i = pl.program_id(0)
    bufs = (b0buf, b1buf, b2buf)

    def _chunk_copy(c, buf, k):
        return pltpu.make_async_copy(
            enc_hbm.at[pl.ds(c * CB, CB), :], buf, sems.at[k]
        )

    @pl.when(i == 0)
    def _prologue():
        _chunk_copy(0, b0buf, 0).start()
        _chunk_copy(1, b1buf, 1).start()
        ar = ar_ref[...]
        intermed = _dot_t(ar, w0_ref[...]) + b0_ref[...]
        intermed = jnp.maximum(
            _dot_t(jnp.maximum(intermed, 0.0), w1_ref[...]) + b1_ref[...], 0.0
        )
        x = jnp.concatenate([intermed, jnp.zeros_like(intermed)], axis=1)
        lnw = lnw_ref[...]
        lnb = lnb_ref[...]
        remember = _ln(
            jax.nn.sigmoid(_dot_t(x, wi0_ref[...]) + bi0_ref[...])
            * jnp.tanh(_dot_t(x, wi1_ref[...]) + bi1_ref[...]),
            lnw, lnb,
        )
        out_gate = _ln(jax.nn.sigmoid(_dot_t(x, wo_ref[...]) + bo_ref[...]), lnw, lnb)
        query = jnp.tanh(remember) * out_gate
        q256 = jax.lax.dot_general(
            query, wk_ref[...], (((1,), (0,)), ((), ())),
            precision=jax.lax.Precision.HIGHEST,
            preferred_element_type=jnp.float32,
        )
        q_sc[0:1, 0:256] = q256
        qb_sc[0] = jnp.sum(query * bk_ref[...])

    q256 = q_sc[0:1, 0:256]

    def _body(k):
        buf = bufs[k]
        _chunk_copy(i, buf, k).wait()

        @pl.when(i + 2 < NCHUNK)
        def _issue_next():
            _chunk_copy(i + 2, bufs[(k + 2) % NBUF], (k + 2) % NBUF).start()

        sim = _dot_t(q256, buf[...]) + qb_sc[0]
        logit = jax.nn.sigmoid(sim)
        vec = jnp.exp(jnp.log(logit) / 0.8)
        row_sc[0:1, pl.ds(i * CB, CB)] = vec

    for k in range(NBUF):
        @pl.when(i % NBUF == k)
        def _run(k=k):
            _body(k)

    @pl.when(i == NCHUNK - 1)
    def _epilogue():
        vecrow = row_sc[...]
        s = jnp.sum(vecrow)
        bmax = jnp.max(vecrow)
        colf = jax.lax.broadcasted_iota(jnp.int32, (1, N_ENT), 1)
        pick = jnp.min(jnp.where(vecrow == bmax, colf, N_ENT))
        unit_ref[...] = jnp.where(s != 0.0, vecrow / s, vecrow)
        targ_ref[...] = jnp.where(
            (colf == pick) & jnp.logical_not(em_ref[...]), 1.0, 0.0
        )


def kernel(utype_mask, entity_mask, entity_encodings, autoregressive_encoding,
           self_unit_ct, W_keys, b_keys, W0, b0, W1, b1, Wf, bf, Wi0, bi0,
           Wi1, bi1, Wo, bo, ln_w, ln_b):
    em = entity_mask.reshape(1, N_ENT)
    ar2 = autoregressive_encoding.reshape(1, 1024)
    row = lambda v: v.reshape(1, -1)

    full = lambda shape: pl.BlockSpec(shape, lambda i: (0, 0))
    unit, targ = pl.pallas_call(
        _fused_kernel,
        grid=(NCHUNK,),
        in_specs=[
            full((1, N_ENT)),
            full((1, 1024)),
            full(W_keys.shape),
            full((1, 32)),
            full(W0.shape), full((1, 256)),
            full(W1.shape), full((1, 32)),
            full(Wf.shape), full((1, 32)),
            full(Wi0.shape), full((1, 32)),
            full(Wi1.shape), full((1, 32)),
            full(Wo.shape), full((1, 32)),
            full((1, 32)), full((1, 32)),
            pl.BlockSpec(memory_space=pltpu.MemorySpace.HBM),
        ],
        out_specs=[
            pl.BlockSpec((1, N_ENT), lambda i: (0, 0)),
            pl.BlockSpec((1, N_ENT), lambda i: (0, 0)),
        ],
        out_shape=[
            jax.ShapeDtypeStruct((1, N_ENT), jnp.float32),
            jax.ShapeDtypeStruct((1, N_ENT), jnp.float32),
        ],
        scratch_shapes=[
            pltpu.VMEM((CB, 256), jnp.float32),
            pltpu.VMEM((CB, 256), jnp.float32),
            pltpu.VMEM((CB, 256), jnp.float32),
            pltpu.VMEM((1, N_ENT), jnp.float32),
            pltpu.VMEM((8, 256), jnp.float32),
            pltpu.SMEM((1,), jnp.float32),
            pltpu.SemaphoreType.DMA((NBUF,)),
        ],
    )(
        em, ar2, W_keys, row(b_keys), W0, row(b0),
        W1, row(b1), Wf, row(bf), Wi0, row(bi0), Wi1, row(bi1),
        Wo, row(bo), row(ln_w), row(ln_b), entity_encodings
    )
    return unit, targ.reshape(N_ENT)
